# Initial kernel scaffold; baseline (speedup 1.0000x reference)
import jax
import jax.numpy as jnp
from jax.experimental import pallas as pl


def kernel(x, edge_index, pre_W, pre_b, s1_Wl, s1_bl, s1_Wr, s2_Wl, s2_bl, s2_Wr, post_W, post_b, out_W, out_b):
    raise NotImplementedError("write your pallas kernel here")


# speedup vs baseline: 2.8053x; 2.8053x over previous
import functools

import jax
import jax.numpy as jnp
from jax import lax
from jax.experimental import pallas as pl
from jax.experimental.pallas import tpu as pltpu
from jax.experimental.pallas import tpu_sc as plsc

N = 10000
E = 320000
F = 128

NC = 2
NS = 16
NTILES = NC * NS

CHUNK = 64
NCHUNK = 160
GSZ = 32
NGROUP = NCHUNK // GSZ
EPT = CHUNK * NCHUNK
E_PAD = EPT * NTILES
N_PAD = 10240
RPT = N_PAD // NS

_BLK = 1000


def _dotT(a, b):
    return lax.dot_general(a, b, (((1,), (1,)), ((), ())),
                           preferred_element_type=jnp.float32)


def _tc1_body(x_ref, w_ref, b_ref, wl_ref, h_ref, t_ref):
    h = jnp.maximum(_dotT(x_ref[...], w_ref[...]) + b_ref[...], 0.0)
    h_ref[...] = h
    t_ref[...] = _dotT(h, wl_ref[...])


def _tc2_body(acc_ref, h_ref, bl_ref, wr_ref, wl2_ref, h1_ref, t1_ref):
    a = acc_ref[0] + acc_ref[1]
    h1 = jnp.maximum(a + bl_ref[...] + _dotT(h_ref[...], wr_ref[...]), 0.0)
    h1_ref[...] = h1
    t1_ref[...] = _dotT(h1, wl2_ref[...])


def _tc3_body(acc_ref, h_ref, bl_ref, wr_ref, pw_ref, pb_ref, ow_ref,
              ob_ref, out_ref):
    a = acc_ref[0] + acc_ref[1]
    h2 = jnp.maximum(a + bl_ref[...] + _dotT(h_ref[...], wr_ref[...]), 0.0)
    h3 = jnp.maximum(_dotT(h2, pw_ref[...]) + pb_ref[...], 0.0)
    out_ref[...] = _dotT(h3, ow_ref[...]) + ob_ref[...]


def _row_spec():
    return pl.BlockSpec((_BLK, F), lambda i: (i, 0))


def _full_spec(shape):
    nd = len(shape)
    return pl.BlockSpec(shape, lambda i: (0,) * nd)


def _acc_spec():
    return pl.BlockSpec((NC, _BLK, F), lambda i: (0, i, 0))


def _tc1(x, w, b, wl):
    return pl.pallas_call(
        _tc1_body,
        grid=(N // _BLK,),
        in_specs=[_row_spec(), _full_spec((F, F)), _full_spec((1, F)),
                  _full_spec((F, F))],
        out_specs=[_row_spec(), _row_spec()],
        out_shape=[jax.ShapeDtypeStruct((N, F), jnp.float32)] * 2,
    )(x, w, b, wl)


def _tc2(acc, h, bl, wr, wl2):
    return pl.pallas_call(
        _tc2_body,
        grid=(N // _BLK,),
        in_specs=[_acc_spec(), _row_spec(), _full_spec((1, F)),
                  _full_spec((F, F)), _full_spec((F, F))],
        out_specs=[_row_spec(), _row_spec()],
        out_shape=[jax.ShapeDtypeStruct((N, F), jnp.float32)] * 2,
    )(acc, h, bl, wr, wl2)


def _tc3(acc, h, bl, wr, pw, pb, ow, ob):
    return pl.pallas_call(
        _tc3_body,
        grid=(N // _BLK,),
        in_specs=[_acc_spec(), _row_spec(), _full_spec((1, F)),
                  _full_spec((F, F)), _full_spec((F, F)), _full_spec((1, F)),
                  _full_spec((F, F)), _full_spec((1, F))],
        out_specs=_row_spec(),
        out_shape=jax.ShapeDtypeStruct((N, F), jnp.float32),
    )(acc, h, bl, wr, pw, pb, ow, ob)


def _seg_body(table_hbm, src_hbm, dst_hbm, out_hbm,
              src_v, dst_v, rows_v, acc_sh, sem, sem_idx):
    c = lax.axis_index("c")
    s = lax.axis_index("s")
    tid = c * NS + s

    zvec = jnp.zeros((16,), jnp.float32)

    def zbody(i, carry):
        for k16 in range(F // 16):
            rows_v[i, pl.ds(k16 * 16, 16)] = zvec
        return carry

    lax.fori_loop(0, 2 * CHUNK, zbody, 0)
    base = s * RPT
    for r in range(RPT // (2 * CHUNK)):
        pltpu.sync_copy(rows_v,
                        acc_sh.at[pl.ds(base + r * 2 * CHUNK, 2 * CHUNK)])
    plsc.subcore_barrier()

    rows0 = rows_v.at[pl.ds(0, CHUNK)]
    rows1 = rows_v.at[pl.ds(CHUNK, CHUNK)]

    pltpu.sync_copy(src_hbm.at[tid, 0], src_v.at[0])
    pltpu.sync_copy(dst_hbm.at[tid, 0], dst_v.at[0])

    for grp in range(NGROUP):
        gb = grp % 2
        nb = (grp + 1) % 2
        if grp + 1 < NGROUP:
            pltpu.make_async_copy(src_hbm.at[tid, grp + 1],
                                  src_v.at[nb], sem_idx).start()
            pltpu.make_async_copy(dst_hbm.at[tid, grp + 1],
                                  dst_v.at[nb], sem_idx).start()

        pltpu.make_async_copy(
            table_hbm.at[src_v.at[gb, 0]], rows0, sem).start()

        def body(jj, carry, gb=gb):
            j0 = 2 * jj
            j1 = j0 + 1
            pltpu.make_async_copy(
                table_hbm.at[src_v.at[gb, j0]], rows0, sem).wait()
            pltpu.make_async_copy(
                table_hbm.at[src_v.at[gb, j1]], rows1, sem).start()
            pltpu.sync_copy(rows0, acc_sh.at[dst_v.at[gb, j0]], add=True)
            pltpu.make_async_copy(
                table_hbm.at[src_v.at[gb, j1]], rows1, sem).wait()

            @pl.when(j1 + 1 < GSZ)
            def _():
                pltpu.make_async_copy(
                    table_hbm.at[src_v.at[gb, j1 + 1]], rows0, sem).start()

            pltpu.sync_copy(rows1, acc_sh.at[dst_v.at[gb, j1]], add=True)
            return carry

        lax.fori_loop(0, GSZ // 2, body, 0)

        if grp + 1 < NGROUP:
            pltpu.make_async_copy(src_hbm.at[tid, grp + 1],
                                  src_v.at[nb], sem_idx).wait()
            pltpu.make_async_copy(dst_hbm.at[tid, grp + 1],
                                  dst_v.at[nb], sem_idx).wait()

    plsc.subcore_barrier()
    pltpu.sync_copy(acc_sh.at[pl.ds(base, RPT)],
                    out_hbm.at[c, pl.ds(base, RPT)])


@functools.cache
def _get_seg_sum():
    return functools.partial(
        pl.kernel,
        out_type=jax.ShapeDtypeStruct((NC, N_PAD, F), jnp.float32),
        mesh=plsc.VectorSubcoreMesh(core_axis_name="c", subcore_axis_name="s",
                                    num_cores=NC, num_subcores=NS),
        scratch_types=[
            pltpu.VMEM((2, GSZ, CHUNK), jnp.int32),
            pltpu.VMEM((2, GSZ, CHUNK), jnp.int32),
            pltpu.VMEM((2 * CHUNK, F), jnp.float32),
            pltpu.VMEM_SHARED((N_PAD, F), jnp.float32),
            pltpu.SemaphoreType.DMA,
            pltpu.SemaphoreType.DMA,
        ],
    )(_seg_body)


def _seg_sum(table, srcg, dstg):
    return _get_seg_sum()(table, srcg, dstg)


def kernel(x, edge_index, pre_W, pre_b, s1_Wl, s1_bl, s1_Wr,
           s2_Wl, s2_bl, s2_Wr, post_W, post_b, out_W, out_b):
    src = edge_index[0]
    dst = edge_index[1]
    pad = E_PAD - E
    srcg = jnp.concatenate(
        [src, jnp.zeros((pad,), jnp.int32)]
    ).reshape(NTILES, NGROUP, GSZ, CHUNK)
    dstg = jnp.concatenate(
        [dst, jnp.full((pad,), N, jnp.int32)]
    ).reshape(NTILES, NGROUP, GSZ, CHUNK)

    pre_b2 = pre_b.reshape(1, F)
    s1_bl2 = s1_bl.reshape(1, F)
    s2_bl2 = s2_bl.reshape(1, F)
    post_b2 = post_b.reshape(1, F)
    out_b2 = out_b.reshape(1, F)

    h0, t0 = _tc1(x, pre_W, pre_b2, s1_Wl)
    acc1 = _seg_sum(t0, srcg, dstg)
    h1, t1 = _tc2(acc1, h0, s1_bl2, s1_Wr, s2_Wl)
    acc2 = _seg_sum(t1, srcg, dstg)
    return _tc3(acc2, h1, s2_bl2, s2_Wr, post_W, post_b2, out_W, out_b2)


# speedup vs baseline: 2.9830x; 1.0634x over previous
import functools

import jax
import jax.numpy as jnp
from jax import lax
from jax.experimental import pallas as pl
from jax.experimental.pallas import tpu as pltpu
from jax.experimental.pallas import tpu_sc as plsc

N = 10000
E = 320000
F = 128

NC = 2
NS = 16
NTILES = NC * NS

CHUNK = 128
NCHUNK = 80
GSZ = 16
NGROUP = NCHUNK // GSZ
EPT = CHUNK * NCHUNK
E_PAD = EPT * NTILES
N_PAD = 10240
RPT = N_PAD // NS

_BLK = 1000


def _dotT(a, b):
    return lax.dot_general(a, b, (((1,), (1,)), ((), ())),
                           preferred_element_type=jnp.float32)


def _tc1_body(x_ref, w_ref, b_ref, wl_ref, h_ref, t_ref):
    h = jnp.maximum(_dotT(x_ref[...], w_ref[...]) + b_ref[...], 0.0)
    h_ref[...] = h
    t_ref[...] = _dotT(h, wl_ref[...])


def _tc2_body(acc_ref, h_ref, bl_ref, wr_ref, wl2_ref, h1_ref, t1_ref):
    a = acc_ref[0] + acc_ref[1]
    h1 = jnp.maximum(a + bl_ref[...] + _dotT(h_ref[...], wr_ref[...]), 0.0)
    h1_ref[...] = h1
    t1_ref[...] = _dotT(h1, wl2_ref[...])


def _tc3_body(acc_ref, h_ref, bl_ref, wr_ref, pw_ref, pb_ref, ow_ref,
              ob_ref, out_ref):
    a = acc_ref[0] + acc_ref[1]
    h2 = jnp.maximum(a + bl_ref[...] + _dotT(h_ref[...], wr_ref[...]), 0.0)
    h3 = jnp.maximum(_dotT(h2, pw_ref[...]) + pb_ref[...], 0.0)
    out_ref[...] = _dotT(h3, ow_ref[...]) + ob_ref[...]


def _row_spec():
    return pl.BlockSpec((_BLK, F), lambda i: (i, 0))


def _full_spec(shape):
    nd = len(shape)
    return pl.BlockSpec(shape, lambda i: (0,) * nd)


def _acc_spec():
    return pl.BlockSpec((NC, _BLK, F), lambda i: (0, i, 0))


def _tc1(x, w, b, wl):
    return pl.pallas_call(
        _tc1_body,
        grid=(N // _BLK,),
        in_specs=[_row_spec(), _full_spec((F, F)), _full_spec((1, F)),
                  _full_spec((F, F))],
        out_specs=[_row_spec(), _row_spec()],
        out_shape=[jax.ShapeDtypeStruct((N, F), jnp.float32)] * 2,
    )(x, w, b, wl)


def _tc2(acc, h, bl, wr, wl2):
    return pl.pallas_call(
        _tc2_body,
        grid=(N // _BLK,),
        in_specs=[_acc_spec(), _row_spec(), _full_spec((1, F)),
                  _full_spec((F, F)), _full_spec((F, F))],
        out_specs=[_row_spec(), _row_spec()],
        out_shape=[jax.ShapeDtypeStruct((N, F), jnp.float32)] * 2,
    )(acc, h, bl, wr, wl2)


def _tc3(acc, h, bl, wr, pw, pb, ow, ob):
    return pl.pallas_call(
        _tc3_body,
        grid=(N // _BLK,),
        in_specs=[_acc_spec(), _row_spec(), _full_spec((1, F)),
                  _full_spec((F, F)), _full_spec((F, F)), _full_spec((1, F)),
                  _full_spec((F, F)), _full_spec((1, F))],
        out_specs=_row_spec(),
        out_shape=jax.ShapeDtypeStruct((N, F), jnp.float32),
    )(acc, h, bl, wr, pw, pb, ow, ob)


def _seg_body(table_hbm, src_hbm, dst_hbm, out_hbm,
              src_v, dst_v, rows_v, acc_sh,
              sem_g0, sem_g1, sem_s0, sem_s1, sem_idx):
    c = lax.axis_index("c")
    s = lax.axis_index("s")
    tid = c * NS + s

    zvec = jnp.zeros((16,), jnp.float32)

    def zbody(i, carry):
        for k16 in range(F // 16):
            rows_v[i, pl.ds(k16 * 16, 16)] = zvec
        return carry

    lax.fori_loop(0, CHUNK, zbody, 0)
    base = s * RPT
    for r in range(RPT // CHUNK):
        pltpu.sync_copy(rows_v.at[pl.ds(0, CHUNK)],
                        acc_sh.at[pl.ds(base + r * CHUNK, CHUNK)])
    plsc.subcore_barrier()

    rows0 = rows_v.at[pl.ds(0, CHUNK)]
    rows1 = rows_v.at[pl.ds(CHUNK, CHUNK)]

    def gather(gb, j, buf, sem):
        return pltpu.make_async_copy(table_hbm.at[src_v.at[gb, j]], buf, sem)

    def scatter(gb, j, buf, sem):
        return pltpu.async_copy(buf, acc_sh.at[dst_v.at[gb, j]], sem,
                                add=True)

    def scatter_wait(gb, j, buf, sem):
        pltpu.make_async_copy(buf, acc_sh.at[dst_v.at[gb, j]], sem).wait()

    pltpu.sync_copy(src_hbm.at[tid, 0], src_v.at[0])
    pltpu.sync_copy(dst_hbm.at[tid, 0], dst_v.at[0])

    for grp in range(NGROUP):
        gb = grp % 2
        nb = (grp + 1) % 2
        if grp + 1 < NGROUP:
            pltpu.make_async_copy(src_hbm.at[tid, grp + 1],
                                  src_v.at[nb], sem_idx).start()
            pltpu.make_async_copy(dst_hbm.at[tid, grp + 1],
                                  dst_v.at[nb], sem_idx).start()

        gather(gb, 0, rows0, sem_g0).start()
        gather(gb, 1, rows1, sem_g1).start()

        def body(jj, carry, gb=gb):
            j0 = 2 * jj
            j1 = j0 + 1
            gather(gb, j0, rows0, sem_g0).wait()
            scatter(gb, j0, rows0, sem_s0)
            gather(gb, j1, rows1, sem_g1).wait()
            scatter(gb, j1, rows1, sem_s1)
            scatter_wait(gb, j0, rows0, sem_s0)

            @pl.when(j0 + 2 < GSZ)
            def _():
                gather(gb, j0 + 2, rows0, sem_g0).start()

            scatter_wait(gb, j1, rows1, sem_s1)

            @pl.when(j1 + 2 < GSZ)
            def _():
                gather(gb, j1 + 2, rows1, sem_g1).start()

            return carry

        lax.fori_loop(0, GSZ // 2, body, 0)

        if grp + 1 < NGROUP:
            pltpu.make_async_copy(src_hbm.at[tid, grp + 1],
                                  src_v.at[nb], sem_idx).wait()
            pltpu.make_async_copy(dst_hbm.at[tid, grp + 1],
                                  dst_v.at[nb], sem_idx).wait()

    plsc.subcore_barrier()
    pltpu.sync_copy(acc_sh.at[pl.ds(base, RPT)],
                    out_hbm.at[c, pl.ds(base, RPT)])


@functools.cache
def _get_seg_sum():
    return functools.partial(
        pl.kernel,
        out_type=jax.ShapeDtypeStruct((NC, N_PAD, F), jnp.float32),
        mesh=plsc.VectorSubcoreMesh(core_axis_name="c", subcore_axis_name="s",
                                    num_cores=NC, num_subcores=NS),
        scratch_types=[
            pltpu.VMEM((2, GSZ, CHUNK), jnp.int32),
            pltpu.VMEM((2, GSZ, CHUNK), jnp.int32),
            pltpu.VMEM((2 * CHUNK, F), jnp.float32),
            pltpu.VMEM_SHARED((N_PAD, F), jnp.float32),
            pltpu.SemaphoreType.DMA,
            pltpu.SemaphoreType.DMA,
            pltpu.SemaphoreType.DMA,
            pltpu.SemaphoreType.DMA,
            pltpu.SemaphoreType.DMA,
        ],
    )(_seg_body)


def _seg_sum(table, srcg, dstg):
    return _get_seg_sum()(table, srcg, dstg)


def kernel(x, edge_index, pre_W, pre_b, s1_Wl, s1_bl, s1_Wr,
           s2_Wl, s2_bl, s2_Wr, post_W, post_b, out_W, out_b):
    src = edge_index[0]
    dst = edge_index[1]
    pad = E_PAD - E
    srcg = jnp.concatenate(
        [src, jnp.zeros((pad,), jnp.int32)]
    ).reshape(NTILES, NGROUP, GSZ, CHUNK)
    dstg = jnp.concatenate(
        [dst, jnp.full((pad,), N, jnp.int32)]
    ).reshape(NTILES, NGROUP, GSZ, CHUNK)

    pre_b2 = pre_b.reshape(1, F)
    s1_bl2 = s1_bl.reshape(1, F)
    s2_bl2 = s2_bl.reshape(1, F)
    post_b2 = post_b.reshape(1, F)
    out_b2 = out_b.reshape(1, F)

    h0, t0 = _tc1(x, pre_W, pre_b2, s1_Wl)
    acc1 = _seg_sum(t0, srcg, dstg)
    h1, t1 = _tc2(acc1, h0, s1_bl2, s1_Wr, s2_Wl)
    acc2 = _seg_sum(t1, srcg, dstg)
    return _tc3(acc2, h1, s2_bl2, s2_Wr, post_W, post_b2, out_W, out_b2)


# speedup vs baseline: 8.8480x; 2.9661x over previous
import functools

import jax
import jax.numpy as jnp
from jax import lax
from jax.experimental import pallas as pl
from jax.experimental.pallas import tpu as pltpu
from jax.experimental.pallas import tpu_sc as plsc

N = 10000
E = 320000
F = 128

NC = 2
NS = 16
NTILES = NC * NS

CHUNK = 128
NCHUNK = 80
GSZ = 16
NGROUP = NCHUNK // GSZ
EPT = CHUNK * NCHUNK
E_PAD = EPT * NTILES
N_PAD = 10240
RPT = N_PAD // NS

_BLK = 1000


def _dotT(a, b):
    return lax.dot_general(a, b, (((1,), (1,)), ((), ())),
                           preferred_element_type=jnp.float32)


def _tc1_body(x_ref, w_ref, b_ref, wl_ref, h_ref, t_ref):
    h = jnp.maximum(_dotT(x_ref[...], w_ref[...]) + b_ref[...], 0.0)
    h_ref[...] = h
    t_ref[...] = _dotT(h, wl_ref[...])


def _tc2_body(acc_ref, h_ref, bl_ref, wr_ref, wl2_ref, h1_ref, t1_ref):
    a = acc_ref[0] + acc_ref[1]
    h1 = jnp.maximum(a + bl_ref[...] + _dotT(h_ref[...], wr_ref[...]), 0.0)
    h1_ref[...] = h1
    t1_ref[...] = _dotT(h1, wl2_ref[...])


def _tc3_body(acc_ref, h_ref, bl_ref, wr_ref, pw_ref, pb_ref, ow_ref,
              ob_ref, out_ref):
    a = acc_ref[0] + acc_ref[1]
    h2 = jnp.maximum(a + bl_ref[...] + _dotT(h_ref[...], wr_ref[...]), 0.0)
    h3 = jnp.maximum(_dotT(h2, pw_ref[...]) + pb_ref[...], 0.0)
    out_ref[...] = _dotT(h3, ow_ref[...]) + ob_ref[...]


def _row_spec():
    return pl.BlockSpec((_BLK, F), lambda i: (i, 0))


def _full_spec(shape):
    nd = len(shape)
    return pl.BlockSpec(shape, lambda i: (0,) * nd)


def _acc_spec():
    return pl.BlockSpec((NC, _BLK, F), lambda i: (0, i, 0))


def _tc1(x, w, b, wl):
    return pl.pallas_call(
        _tc1_body,
        grid=(N // _BLK,),
        in_specs=[_row_spec(), _full_spec((F, F)), _full_spec((1, F)),
                  _full_spec((F, F))],
        out_specs=[_row_spec(), _row_spec()],
        out_shape=[jax.ShapeDtypeStruct((N, F), jnp.float32)] * 2,
    )(x, w, b, wl)


def _tc2(acc, h, bl, wr, wl2):
    return pl.pallas_call(
        _tc2_body,
        grid=(N // _BLK,),
        in_specs=[_acc_spec(), _row_spec(), _full_spec((1, F)),
                  _full_spec((F, F)), _full_spec((F, F))],
        out_specs=[_row_spec(), _row_spec()],
        out_shape=[jax.ShapeDtypeStruct((N, F), jnp.float32)] * 2,
    )(acc, h, bl, wr, wl2)


def _tc3(acc, h, bl, wr, pw, pb, ow, ob):
    return pl.pallas_call(
        _tc3_body,
        grid=(N // _BLK,),
        in_specs=[_acc_spec(), _row_spec(), _full_spec((1, F)),
                  _full_spec((F, F)), _full_spec((F, F)), _full_spec((1, F)),
                  _full_spec((F, F)), _full_spec((1, F))],
        out_specs=_row_spec(),
        out_shape=jax.ShapeDtypeStruct((N, F), jnp.float32),
    )(acc, h, bl, wr, pw, pb, ow, ob)


def _seg_body(table_hbm, src_hbm, dst_hbm, out_hbm,
              src_v, dst_v, rows_v, acc_sh,
              sem_g0, sem_g1, sem_s0, sem_s1, sem_idx):
    c = lax.axis_index("c")
    s = lax.axis_index("s")
    tid = c * NS + s

    zvec = jnp.zeros((16,), jnp.float32)

    def zbody(i, carry):
        for k16 in range(F // 16):
            rows_v[i, pl.ds(k16 * 16, 16)] = zvec
        return carry

    lax.fori_loop(0, CHUNK, zbody, 0)
    base = s * RPT
    for r in range(RPT // CHUNK):
        pltpu.sync_copy(rows_v.at[pl.ds(0, CHUNK)],
                        acc_sh.at[pl.ds(base + r * CHUNK, CHUNK)])
    plsc.subcore_barrier()

    rows0 = rows_v.at[pl.ds(0, CHUNK)]
    rows1 = rows_v.at[pl.ds(CHUNK, CHUNK)]

    def gather(gb, j, buf, sem):
        return pltpu.make_async_copy(table_hbm.at[src_v.at[gb, j]], buf, sem)

    def scatter(gb, j, buf, sem):
        return pltpu.async_copy(buf, acc_sh.at[dst_v.at[gb, j]], sem,
                                add=True)

    def scatter_wait(gb, j, buf, sem):
        pltpu.make_async_copy(buf, acc_sh.at[dst_v.at[gb, j]], sem).wait()

    pltpu.sync_copy(src_hbm.at[tid, 0], src_v.at[0])
    pltpu.sync_copy(dst_hbm.at[tid, 0], dst_v.at[0])

    for grp in range(NGROUP):
        gb = grp % 2
        nb = (grp + 1) % 2
        if grp + 1 < NGROUP:
            pltpu.make_async_copy(src_hbm.at[tid, grp + 1],
                                  src_v.at[nb], sem_idx).start()
            pltpu.make_async_copy(dst_hbm.at[tid, grp + 1],
                                  dst_v.at[nb], sem_idx).start()

        gather(gb, 0, rows0, sem_g0).start()
        gather(gb, 1, rows1, sem_g1).start()

        def body(jj, carry, gb=gb):
            j0 = 2 * jj
            j1 = j0 + 1
            gather(gb, j0, rows0, sem_g0).wait()
            scatter(gb, j0, rows0, sem_s0)
            gather(gb, j1, rows1, sem_g1).wait()
            scatter(gb, j1, rows1, sem_s1)
            scatter_wait(gb, j0, rows0, sem_s0)

            @pl.when(j0 + 2 < GSZ)
            def _():
                gather(gb, j0 + 2, rows0, sem_g0).start()

            scatter_wait(gb, j1, rows1, sem_s1)

            @pl.when(j1 + 2 < GSZ)
            def _():
                gather(gb, j1 + 2, rows1, sem_g1).start()

            return carry

        lax.fori_loop(0, GSZ // 2, body, 0)

        if grp + 1 < NGROUP:
            pltpu.make_async_copy(src_hbm.at[tid, grp + 1],
                                  src_v.at[nb], sem_idx).wait()
            pltpu.make_async_copy(dst_hbm.at[tid, grp + 1],
                                  dst_v.at[nb], sem_idx).wait()

    plsc.subcore_barrier()
    pltpu.sync_copy(acc_sh.at[pl.ds(base, RPT)],
                    out_hbm.at[c, pl.ds(base, RPT)])


@functools.cache
def _get_seg_sum():
    return functools.partial(
        pl.kernel,
        out_type=jax.ShapeDtypeStruct((NC, N_PAD, F), jnp.float32),
        mesh=plsc.VectorSubcoreMesh(core_axis_name="c", subcore_axis_name="s",
                                    num_cores=NC, num_subcores=NS),
        scratch_types=[
            pltpu.VMEM((2, GSZ, CHUNK), jnp.int32),
            pltpu.VMEM((2, GSZ, CHUNK), jnp.int32),
            pltpu.VMEM((2 * CHUNK, F), jnp.float32),
            pltpu.VMEM_SHARED((N_PAD, F), jnp.float32),
            pltpu.SemaphoreType.DMA,
            pltpu.SemaphoreType.DMA,
            pltpu.SemaphoreType.DMA,
            pltpu.SemaphoreType.DMA,
            pltpu.SemaphoreType.DMA,
        ],
    )(_seg_body)


def _seg_sum(table, srcg, dstg):
    return _get_seg_sum()(table, srcg, dstg)


def kernel(x, edge_index, pre_W, pre_b, s1_Wl, s1_bl, s1_Wr,
           s2_Wl, s2_bl, s2_Wr, post_W, post_b, out_W, out_b):
    src = edge_index[0]
    dst = edge_index[1]
    pad = E_PAD - E
    pad_iota = jnp.arange(pad, dtype=jnp.int32)
    srcg = jnp.concatenate(
        [src, pad_iota % N]
    ).reshape(NTILES, NGROUP, GSZ, CHUNK)
    dstg = jnp.concatenate(
        [dst, N + pad_iota % (N_PAD - N)]
    ).reshape(NTILES, NGROUP, GSZ, CHUNK)

    pre_b2 = pre_b.reshape(1, F)
    s1_bl2 = s1_bl.reshape(1, F)
    s2_bl2 = s2_bl.reshape(1, F)
    post_b2 = post_b.reshape(1, F)
    out_b2 = out_b.reshape(1, F)

    h0, t0 = _tc1(x, pre_W, pre_b2, s1_Wl)
    acc1 = _seg_sum(t0, srcg, dstg)
    h1, t1 = _tc2(acc1, h0, s1_bl2, s1_Wr, s2_Wl)
    acc2 = _seg_sum(t1, srcg, dstg)
    return _tc3(acc2, h1, s2_bl2, s2_Wr, post_W, post_b2, out_W, out_b2)


# speedup vs baseline: 9.9282x; 1.1221x over previous
import functools

import jax
import jax.numpy as jnp
from jax import lax
from jax.experimental import pallas as pl
from jax.experimental.pallas import tpu as pltpu
from jax.experimental.pallas import tpu_sc as plsc

N = 10000
E = 320000
F = 128

NC = 2
NS = 16
NTILES = NC * NS

CHUNK = 64
NCHUNK = 160
GSZ = 32
NGROUP = NCHUNK // GSZ
NBUF = 4
EPT = CHUNK * NCHUNK
E_PAD = EPT * NTILES
N_PAD = 10240
RPT = N_PAD // NS

_BLK = 1000


def _dotT(a, b):
    return lax.dot_general(a, b, (((1,), (1,)), ((), ())),
                           preferred_element_type=jnp.float32)


def _tc1_body(x_ref, w_ref, b_ref, wl_ref, h_ref, t_ref):
    h = jnp.maximum(_dotT(x_ref[...], w_ref[...]) + b_ref[...], 0.0)
    h_ref[...] = h
    t_ref[...] = _dotT(h, wl_ref[...])


def _tc2_body(acc_ref, h_ref, bl_ref, wr_ref, wl2_ref, h1_ref, t1_ref):
    a = acc_ref[0] + acc_ref[1]
    h1 = jnp.maximum(a + bl_ref[...] + _dotT(h_ref[...], wr_ref[...]), 0.0)
    h1_ref[...] = h1
    t1_ref[...] = _dotT(h1, wl2_ref[...])


def _tc3_body(acc_ref, h_ref, bl_ref, wr_ref, pw_ref, pb_ref, ow_ref,
              ob_ref, out_ref):
    a = acc_ref[0] + acc_ref[1]
    h2 = jnp.maximum(a + bl_ref[...] + _dotT(h_ref[...], wr_ref[...]), 0.0)
    h3 = jnp.maximum(_dotT(h2, pw_ref[...]) + pb_ref[...], 0.0)
    out_ref[...] = _dotT(h3, ow_ref[...]) + ob_ref[...]


def _row_spec():
    return pl.BlockSpec((_BLK, F), lambda i: (i, 0))


def _full_spec(shape):
    nd = len(shape)
    return pl.BlockSpec(shape, lambda i: (0,) * nd)


def _acc_spec():
    return pl.BlockSpec((NC, _BLK, F), lambda i: (0, i, 0))


def _tc1(x, w, b, wl):
    return pl.pallas_call(
        _tc1_body,
        grid=(N // _BLK,),
        in_specs=[_row_spec(), _full_spec((F, F)), _full_spec((1, F)),
                  _full_spec((F, F))],
        out_specs=[_row_spec(), _row_spec()],
        out_shape=[jax.ShapeDtypeStruct((N, F), jnp.float32)] * 2,
    )(x, w, b, wl)


def _tc2(acc, h, bl, wr, wl2):
    return pl.pallas_call(
        _tc2_body,
        grid=(N // _BLK,),
        in_specs=[_acc_spec(), _row_spec(), _full_spec((1, F)),
                  _full_spec((F, F)), _full_spec((F, F))],
        out_specs=[_row_spec(), _row_spec()],
        out_shape=[jax.ShapeDtypeStruct((N, F), jnp.float32)] * 2,
    )(acc, h, bl, wr, wl2)


def _tc3(acc, h, bl, wr, pw, pb, ow, ob):
    return pl.pallas_call(
        _tc3_body,
        grid=(N // _BLK,),
        in_specs=[_acc_spec(), _row_spec(), _full_spec((1, F)),
                  _full_spec((F, F)), _full_spec((F, F)), _full_spec((1, F)),
                  _full_spec((F, F)), _full_spec((1, F))],
        out_specs=_row_spec(),
        out_shape=jax.ShapeDtypeStruct((N, F), jnp.float32),
    )(acc, h, bl, wr, pw, pb, ow, ob)


def _seg_body(table_hbm, src_hbm, dst_hbm, out_hbm,
              src_v, dst_v, rows_v, acc_sh,
              sem_g0, sem_g1, sem_g2, sem_g3,
              sem_s0, sem_s1, sem_s2, sem_s3, sem_idx):
    c = lax.axis_index("c")
    s = lax.axis_index("s")
    tid = c * NS + s

    zvec = jnp.zeros((16,), jnp.float32)

    def zbody(i, carry):
        for k16 in range(F // 16):
            rows_v[i, pl.ds(k16 * 16, 16)] = zvec
        return carry

    zrows = NBUF * CHUNK
    lax.fori_loop(0, zrows, zbody, 0)
    base = s * RPT
    for r in range(RPT // zrows):
        pltpu.sync_copy(rows_v,
                        acc_sh.at[pl.ds(base + r * zrows, zrows)])
    pltpu.sync_copy(rows_v.at[pl.ds(0, RPT % zrows)],
                    acc_sh.at[pl.ds(base + RPT - RPT % zrows, RPT % zrows)])
    plsc.subcore_barrier()

    bufs = [rows_v.at[pl.ds(k * CHUNK, CHUNK)] for k in range(NBUF)]

    def gather(gb, j, k, sem):
        return pltpu.make_async_copy(table_hbm.at[src_v.at[gb, j]],
                                     bufs[k], sem)

    def scatter(gb, j, k, sem):
        return pltpu.async_copy(bufs[k], acc_sh.at[dst_v.at[gb, j]], sem,
                                add=True)

    def scatter_wait(gb, j, k, sem):
        pltpu.make_async_copy(bufs[k], acc_sh.at[dst_v.at[gb, j]],
                              sem).wait()

    sem_g = [sem_g0, sem_g1, sem_g2, sem_g3]
    sem_s = [sem_s0, sem_s1, sem_s2, sem_s3]

    pltpu.sync_copy(src_hbm.at[tid, 0], src_v.at[0])
    pltpu.sync_copy(dst_hbm.at[tid, 0], dst_v.at[0])

    for grp in range(NGROUP):
        gb = grp % 2
        nb = (grp + 1) % 2
        if grp + 1 < NGROUP:
            pltpu.make_async_copy(src_hbm.at[tid, grp + 1],
                                  src_v.at[nb], sem_idx).start()
            pltpu.make_async_copy(dst_hbm.at[tid, grp + 1],
                                  dst_v.at[nb], sem_idx).start()

        gather(gb, 0, 0, sem_g[0]).start()
        gather(gb, 1, 1, sem_g[1]).start()

        def body(qq, carry, gb=gb):
            for k in range(NBUF):
                j = 4 * qq + k
                ka = (k + 2) % NBUF
                gather(gb, j, k, sem_g[k]).wait()
                scatter(gb, j, k, sem_s[k])

                @pl.when(j + 2 < GSZ)
                def _(j=j, k=k, ka=ka):
                    @pl.when(j >= 2)
                    def _():
                        scatter_wait(gb, j - 2, ka, sem_s[ka])

                    gather(gb, j + 2, ka, sem_g[ka]).start()

            return carry

        lax.fori_loop(0, GSZ // NBUF, body, 0)

        for j in range(GSZ - NBUF, GSZ):
            scatter_wait(gb, j, j % NBUF, sem_s[j % NBUF])

        if grp + 1 < NGROUP:
            pltpu.make_async_copy(src_hbm.at[tid, grp + 1],
                                  src_v.at[nb], sem_idx).wait()
            pltpu.make_async_copy(dst_hbm.at[tid, grp + 1],
                                  dst_v.at[nb], sem_idx).wait()

    plsc.subcore_barrier()
    pltpu.sync_copy(acc_sh.at[pl.ds(base, RPT)],
                    out_hbm.at[c, pl.ds(base, RPT)])


@functools.cache
def _get_seg_sum():
    return functools.partial(
        pl.kernel,
        out_type=jax.ShapeDtypeStruct((NC, N_PAD, F), jnp.float32),
        mesh=plsc.VectorSubcoreMesh(core_axis_name="c", subcore_axis_name="s",
                                    num_cores=NC, num_subcores=NS),
        scratch_types=[
            pltpu.VMEM((2, GSZ, CHUNK), jnp.int32),
            pltpu.VMEM((2, GSZ, CHUNK), jnp.int32),
            pltpu.VMEM((NBUF * CHUNK, F), jnp.float32),
            pltpu.VMEM_SHARED((N_PAD, F), jnp.float32),
        ] + [pltpu.SemaphoreType.DMA] * (2 * NBUF + 1),
    )(_seg_body)


def _seg_sum(table, srcg, dstg):
    return _get_seg_sum()(table, srcg, dstg)


def kernel(x, edge_index, pre_W, pre_b, s1_Wl, s1_bl, s1_Wr,
           s2_Wl, s2_bl, s2_Wr, post_W, post_b, out_W, out_b):
    src = edge_index[0]
    dst = edge_index[1]
    pad = E_PAD - E
    pad_iota = jnp.arange(pad, dtype=jnp.int32)
    srcg = jnp.concatenate(
        [src, pad_iota % N]
    ).reshape(NTILES, NGROUP, GSZ, CHUNK)
    dstg = jnp.concatenate(
        [dst, N + pad_iota % (N_PAD - N)]
    ).reshape(NTILES, NGROUP, GSZ, CHUNK)

    pre_b2 = pre_b.reshape(1, F)
    s1_bl2 = s1_bl.reshape(1, F)
    s2_bl2 = s2_bl.reshape(1, F)
    post_b2 = post_b.reshape(1, F)
    out_b2 = out_b.reshape(1, F)

    h0, t0 = _tc1(x, pre_W, pre_b2, s1_Wl)
    acc1 = _seg_sum(t0, srcg, dstg)
    h1, t1 = _tc2(acc1, h0, s1_bl2, s1_Wr, s2_Wl)
    acc2 = _seg_sum(t1, srcg, dstg)
    return _tc3(acc2, h1, s2_bl2, s2_Wr, post_W, post_b2, out_W, out_b2)
